# baseline (speedup 1.0000x reference)
import functools

import jax
import jax.numpy as jnp
from jax import lax
from jax.experimental import pallas as pl
from jax.experimental.pallas import tpu as pltpu
from jax.experimental.pallas import tpu_sc as plsc

N = 10000
E = 160000
CHUNK = 128
EB = 64
NUM_SC = 2
NUM_TILES = 16
BPT = 160
EP = NUM_TILES * BPT * EB
PADN = BPT * EB - E // NUM_TILES
DUMP = 16
NPAD = N + DUMP
HBPT = BPT // 4


def _sc_agg(h_ch, src, dst):
    C = h_ch.shape[0]
    chunks_per_core = C // NUM_SC
    mesh = plsc.VectorSubcoreMesh(core_axis_name="c", subcore_axis_name="s")

    @functools.partial(
        pl.kernel,
        out_type=jax.ShapeDtypeStruct((C, N, CHUNK), jnp.float32),
        mesh=mesh,
        scratch_types=[
            pltpu.VMEM_SHARED((NPAD, CHUNK), jnp.float32),
            pltpu.VMEM((HBPT, EB), jnp.int32),
            pltpu.VMEM((HBPT, EB), jnp.int32),
            pltpu.VMEM((EB, CHUNK), jnp.float32),
            pltpu.VMEM((EB, CHUNK), jnp.float32),
            pltpu.VMEM((EB, CHUNK), jnp.float32),
            pltpu.VMEM((EB, CHUNK), jnp.float32),
            pltpu.SemaphoreType.DMA,
            pltpu.SemaphoreType.DMA,
            pltpu.SemaphoreType.DMA,
            pltpu.SemaphoreType.DMA,
            pltpu.SemaphoreType.DMA,
            pltpu.SemaphoreType.DMA,
            pltpu.SemaphoreType.DMA,
            pltpu.SemaphoreType.DMA,
        ],
    )
    def agg_kernel(h_hbm, src_hbm, dst_hbm, z_hbm, acc,
                   srcall, dstall, r0, r1, r2, r3,
                   g0, g1, g2, g3, s0, s1, s2, s3):
        rows = [r0, r1, r2, r3]
        gsem = [g0, g1, g2, g3]
        ssem = [s0, s1, s2, s3]
        cid = lax.axis_index("c")
        sid = lax.axis_index("s")
        rpt = 624
        tail0 = rpt * NUM_TILES
        tail = N - tail0
        row0 = sid * rpt

        for chunk in range(C):
            owner = chunk // chunks_per_core
            hc = h_hbm.at[chunk]

            @pl.when(cid == owner)
            def _():
                pltpu.sync_copy(
                    h_hbm.at[chunk, pl.ds(row0, rpt)],
                    acc.at[pl.ds(row0, rpt)],
                )

                @pl.when(sid == 0)
                def _():
                    pltpu.sync_copy(
                        h_hbm.at[chunk, pl.ds(tail0, tail)],
                        acc.at[pl.ds(tail0, tail)],
                    )

                plsc.subcore_barrier()

                def start_gather(k, b):
                    pltpu.async_copy(hc.at[srcall.at[k]], rows[b], gsem[b])

                def wait_gather(k, b):
                    pltpu.make_async_copy(hc.at[srcall.at[k]], rows[b],
                                          gsem[b]).wait()

                def start_scatter(k, b):
                    pltpu.async_copy(rows[b], acc.at[dstall.at[k]], ssem[b],
                                     add=True)

                def wait_scatter(k, b):
                    pltpu.make_async_copy(rows[b], acc.at[dstall.at[k]],
                                          ssem[b]).wait()

                for half in range(4):
                    pltpu.sync_copy(
                        src_hbm.at[pl.ds(sid * BPT + half * HBPT, HBPT)],
                        srcall)
                    pltpu.sync_copy(
                        dst_hbm.at[pl.ds(sid * BPT + half * HBPT, HBPT)],
                        dstall)
                    start_gather(0, 0)
                    start_gather(1, 1)

                    @pl.loop(0, HBPT, step=4)
                    def _(k):
                        for b in range(4):
                            kb = k + b
                            wait_gather(kb, b)
                            start_scatter(kb, b)
                            nb = (b + 2) % 4

                            @pl.when(kb + 2 < HBPT)
                            def _():
                                @pl.when(kb >= 2)
                                def _():
                                    wait_scatter(kb - 2, nb)

                                start_gather(kb + 2, nb)

                    for b in range(4):
                        wait_scatter(HBPT - 4 + b, b)

                plsc.subcore_barrier()

                pltpu.sync_copy(
                    acc.at[pl.ds(row0, rpt)],
                    z_hbm.at[chunk, pl.ds(row0, rpt)],
                )

                @pl.when(sid == 0)
                def _():
                    pltpu.sync_copy(
                        acc.at[pl.ds(tail0, tail)],
                        z_hbm.at[chunk, pl.ds(tail0, tail)],
                    )

                plsc.subcore_barrier()

    return agg_kernel(h_ch, src, dst)


def _mlp_body(z_ref, w1_ref, b1_ref, w2_ref, b2_ref, out_ref, *, relu_out):
    zc = z_ref[...]
    z = jnp.concatenate([zc[c] for c in range(zc.shape[0])], axis=-1)
    a = jnp.maximum(
        jnp.dot(z, w1_ref[...], preferred_element_type=jnp.float32) + b1_ref[...],
        0.0,
    )
    y = jnp.dot(a, w2_ref[...], preferred_element_type=jnp.float32) + b2_ref[...]
    if relu_out:
        y = jnp.maximum(y, 0.0)
    for c in range(out_ref.shape[0]):
        out_ref[c] = y[:, c * CHUNK:(c + 1) * CHUNK]


def _tc_mlp(z_ch, w1, b1, w2, b2, *, bn=1000, relu_out=True):
    C = z_ch.shape[0]
    H = w2.shape[1]
    Co = H // CHUNK
    grid = (N // bn,)
    return pl.pallas_call(
        functools.partial(_mlp_body, relu_out=relu_out),
        grid=grid,
        in_specs=[
            pl.BlockSpec((C, bn, CHUNK), lambda i: (0, i, 0)),
            pl.BlockSpec((C * CHUNK, H), lambda i: (0, 0)),
            pl.BlockSpec((1, H), lambda i: (0, 0)),
            pl.BlockSpec((H, H), lambda i: (0, 0)),
            pl.BlockSpec((1, H), lambda i: (0, 0)),
        ],
        out_specs=pl.BlockSpec((Co, bn, CHUNK), lambda i: (0, i, 0)),
        out_shape=jax.ShapeDtypeStruct((Co, N, CHUNK), jnp.float32),
    )(z_ch, w1, b1.reshape(1, H), w2, b2.reshape(1, H))


def _head_body(z_ref, w1_ref, b1_ref, w2_ref, b2_ref, wh_ref, bh_ref,
               out_ref, acc_ref):
    i = pl.program_id(0)
    zc = z_ref[...]
    z = jnp.concatenate([zc[c] for c in range(zc.shape[0])], axis=-1)
    a = jnp.maximum(
        jnp.dot(z, w1_ref[...], preferred_element_type=jnp.float32) + b1_ref[...],
        0.0,
    )
    y = jnp.dot(a, w2_ref[...], preferred_element_type=jnp.float32) + b2_ref[...]
    colsum = jnp.sum(y, axis=0, keepdims=True)

    @pl.when(i == 0)
    def _():
        acc_ref[...] = jnp.zeros_like(acc_ref)

    acc_ref[...] += colsum
    out_ref[...] = (
        jnp.dot(acc_ref[...] / N, wh_ref[...], preferred_element_type=jnp.float32)
        + bh_ref[...]
    )


def _tc_mlp_head(z_ch, w1, b1, w2, b2, wh, bh, *, bn=1000):
    C = z_ch.shape[0]
    H = w2.shape[1]
    OUT = wh.shape[1]
    grid = (N // bn,)
    return pl.pallas_call(
        _head_body,
        grid=grid,
        in_specs=[
            pl.BlockSpec((C, bn, CHUNK), lambda i: (0, i, 0)),
            pl.BlockSpec((C * CHUNK, H), lambda i: (0, 0)),
            pl.BlockSpec((1, H), lambda i: (0, 0)),
            pl.BlockSpec((H, H), lambda i: (0, 0)),
            pl.BlockSpec((1, H), lambda i: (0, 0)),
            pl.BlockSpec((H, OUT), lambda i: (0, 0)),
            pl.BlockSpec((1, OUT), lambda i: (0, 0)),
        ],
        out_specs=pl.BlockSpec((1, OUT), lambda i: (0, 0)),
        out_shape=jax.ShapeDtypeStruct((1, OUT), jnp.float32),
        scratch_shapes=[pltpu.VMEM((1, H), jnp.float32)],
    )(z_ch, w1, b1.reshape(1, H), w2, b2.reshape(1, H), wh, bh.reshape(1, OUT))


@jax.jit
def kernel(x, edge_index, params):
    ept = E // NUM_TILES
    srcr = edge_index[0].reshape(NUM_TILES, ept)
    dstr = edge_index[1].reshape(NUM_TILES, ept)
    pad_dst = N + jnp.broadcast_to(jnp.arange(PADN, dtype=jnp.int32) % DUMP,
                                   (NUM_TILES, PADN))
    pad_src = jnp.broadcast_to(jnp.arange(PADN, dtype=jnp.int32),
                               (NUM_TILES, PADN))
    src = jnp.concatenate([srcr, pad_src], axis=1).reshape(NUM_TILES * BPT, EB)
    dst = jnp.concatenate([dstr, pad_dst], axis=1).reshape(NUM_TILES * BPT, EB)
    L = len(params["convs"])

    IN = x.shape[1]
    h = jnp.moveaxis(x.reshape(N, IN // CHUNK, CHUNK), 1, 0)

    out = None
    for l in range(L):
        p = params["convs"][l]
        z = _sc_agg(h, src, dst)
        if l < L - 1:
            h = _tc_mlp(z, p["W1"], p["b1"], p["W2"], p["b2"], relu_out=True)
        else:
            hp = params["head"]
            out = _tc_mlp_head(z, p["W1"], p["b1"], p["W2"], p["b2"],
                               hp["Wh"], hp["bh"])
    return out


# speedup vs baseline: 1.1528x; 1.1528x over previous
import functools

import jax
import jax.numpy as jnp
from jax import lax
from jax.experimental import pallas as pl
from jax.experimental.pallas import tpu as pltpu
from jax.experimental.pallas import tpu_sc as plsc

N = 10000
E = 160000
CHUNK = 128
EB = 128
NUM_SC = 2
NUM_TILES = 16
BPT = 80
EP = NUM_TILES * BPT * EB
PADN = BPT * EB - E // NUM_TILES
DUMP = 16
NPAD = N + DUMP
HBPT = BPT // 2


def _sc_agg(h_ch, src, dst):
    C = h_ch.shape[0]
    chunks_per_core = C // NUM_SC
    mesh = plsc.VectorSubcoreMesh(core_axis_name="c", subcore_axis_name="s")

    @functools.partial(
        pl.kernel,
        out_type=jax.ShapeDtypeStruct((C, N, CHUNK), jnp.float32),
        mesh=mesh,
        scratch_types=[
            pltpu.VMEM_SHARED((NPAD, CHUNK), jnp.float32),
            pltpu.VMEM((HBPT, EB), jnp.int32),
            pltpu.VMEM((HBPT, EB), jnp.int32),
            pltpu.VMEM((EB, CHUNK), jnp.float32),
            pltpu.VMEM((EB, CHUNK), jnp.float32),
            pltpu.SemaphoreType.DMA,
            pltpu.SemaphoreType.DMA,
            pltpu.SemaphoreType.DMA,
            pltpu.SemaphoreType.DMA,
        ],
    )
    def agg_kernel(h_hbm, src_hbm, dst_hbm, z_hbm, acc,
                   srcall, dstall, r0, r1, g0, g1, s0, s1):
        rows = [r0, r1]
        gsem = [g0, g1]
        ssem = [s0, s1]
        cid = lax.axis_index("c")
        sid = lax.axis_index("s")
        rpt = 624
        tail0 = rpt * NUM_TILES
        tail = N - tail0
        row0 = sid * rpt

        for chunk in range(C):
            owner = chunk // chunks_per_core
            hc = h_hbm.at[chunk]

            @pl.when(cid == owner)
            def _():
                pltpu.sync_copy(
                    h_hbm.at[chunk, pl.ds(row0, rpt)],
                    acc.at[pl.ds(row0, rpt)],
                )

                @pl.when(sid == 0)
                def _():
                    pltpu.sync_copy(
                        h_hbm.at[chunk, pl.ds(tail0, tail)],
                        acc.at[pl.ds(tail0, tail)],
                    )

                plsc.subcore_barrier()

                def start_gather(k, b):
                    pltpu.async_copy(hc.at[srcall.at[k]], rows[b], gsem[b])

                def wait_gather(k, b):
                    pltpu.make_async_copy(hc.at[srcall.at[k]], rows[b],
                                          gsem[b]).wait()

                def start_scatter(k, b):
                    pltpu.async_copy(rows[b], acc.at[dstall.at[k]], ssem[b],
                                     add=True)

                def wait_scatter(k, b):
                    pltpu.make_async_copy(rows[b], acc.at[dstall.at[k]],
                                          ssem[b]).wait()

                for half in range(2):
                    pltpu.sync_copy(
                        src_hbm.at[pl.ds(sid * BPT + half * HBPT, HBPT)],
                        srcall)
                    pltpu.sync_copy(
                        dst_hbm.at[pl.ds(sid * BPT + half * HBPT, HBPT)],
                        dstall)
                    start_gather(0, 0)

                    @pl.loop(0, HBPT, step=2)
                    def _(k):
                        @pl.when(k > 0)
                        def _():
                            wait_scatter(k - 1, 1)

                        start_gather(k + 1, 1)
                        wait_gather(k, 0)
                        start_scatter(k, 0)

                        @pl.when(k + 2 < HBPT)
                        def _():
                            wait_scatter(k, 0)
                            start_gather(k + 2, 0)

                        wait_gather(k + 1, 1)
                        start_scatter(k + 1, 1)

                    wait_scatter(HBPT - 2, 0)
                    wait_scatter(HBPT - 1, 1)

                plsc.subcore_barrier()

                pltpu.sync_copy(
                    acc.at[pl.ds(row0, rpt)],
                    z_hbm.at[chunk, pl.ds(row0, rpt)],
                )

                @pl.when(sid == 0)
                def _():
                    pltpu.sync_copy(
                        acc.at[pl.ds(tail0, tail)],
                        z_hbm.at[chunk, pl.ds(tail0, tail)],
                    )

                plsc.subcore_barrier()

    return agg_kernel(h_ch, src, dst)


def _mlp_body(z_ref, w1_ref, b1_ref, w2_ref, b2_ref, out_ref, *, relu_out):
    zc = z_ref[...]
    z = jnp.concatenate([zc[c] for c in range(zc.shape[0])], axis=-1)
    a = jnp.maximum(
        jnp.dot(z.astype(jnp.bfloat16), w1_ref[...],
                preferred_element_type=jnp.float32) + b1_ref[...],
        0.0,
    )
    y = jnp.dot(a.astype(jnp.bfloat16), w2_ref[...],
                preferred_element_type=jnp.float32) + b2_ref[...]
    if relu_out:
        y = jnp.maximum(y, 0.0)
    for c in range(out_ref.shape[0]):
        out_ref[c] = y[:, c * CHUNK:(c + 1) * CHUNK]


def _tc_mlp(z_ch, w1, b1, w2, b2, *, bn=1000, relu_out=True):
    C = z_ch.shape[0]
    H = w2.shape[1]
    Co = H // CHUNK
    grid = (N // bn,)
    return pl.pallas_call(
        functools.partial(_mlp_body, relu_out=relu_out),
        grid=grid,
        in_specs=[
            pl.BlockSpec((C, bn, CHUNK), lambda i: (0, i, 0)),
            pl.BlockSpec((C * CHUNK, H), lambda i: (0, 0)),
            pl.BlockSpec((1, H), lambda i: (0, 0)),
            pl.BlockSpec((H, H), lambda i: (0, 0)),
            pl.BlockSpec((1, H), lambda i: (0, 0)),
        ],
        out_specs=pl.BlockSpec((Co, bn, CHUNK), lambda i: (0, i, 0)),
        out_shape=jax.ShapeDtypeStruct((Co, N, CHUNK), jnp.float32),
    )(z_ch, w1.astype(jnp.bfloat16), b1.reshape(1, H),
      w2.astype(jnp.bfloat16), b2.reshape(1, H))


def _head_body(z_ref, w1_ref, b1_ref, w2_ref, b2_ref, wh_ref, bh_ref,
               out_ref, acc_ref):
    i = pl.program_id(0)
    zc = z_ref[...]
    z = jnp.concatenate([zc[c] for c in range(zc.shape[0])], axis=-1)
    a = jnp.maximum(
        jnp.dot(z.astype(jnp.bfloat16), w1_ref[...],
                preferred_element_type=jnp.float32) + b1_ref[...],
        0.0,
    )
    y = jnp.dot(a.astype(jnp.bfloat16), w2_ref[...],
                preferred_element_type=jnp.float32) + b2_ref[...]
    colsum = jnp.sum(y, axis=0, keepdims=True)

    @pl.when(i == 0)
    def _():
        acc_ref[...] = jnp.zeros_like(acc_ref)

    acc_ref[...] += colsum
    out_ref[...] = (
        jnp.dot(acc_ref[...] / N, wh_ref[...], preferred_element_type=jnp.float32)
        + bh_ref[...]
    )


def _tc_mlp_head(z_ch, w1, b1, w2, b2, wh, bh, *, bn=1000):
    C = z_ch.shape[0]
    H = w2.shape[1]
    OUT = wh.shape[1]
    grid = (N // bn,)
    return pl.pallas_call(
        _head_body,
        grid=grid,
        in_specs=[
            pl.BlockSpec((C, bn, CHUNK), lambda i: (0, i, 0)),
            pl.BlockSpec((C * CHUNK, H), lambda i: (0, 0)),
            pl.BlockSpec((1, H), lambda i: (0, 0)),
            pl.BlockSpec((H, H), lambda i: (0, 0)),
            pl.BlockSpec((1, H), lambda i: (0, 0)),
            pl.BlockSpec((H, OUT), lambda i: (0, 0)),
            pl.BlockSpec((1, OUT), lambda i: (0, 0)),
        ],
        out_specs=pl.BlockSpec((1, OUT), lambda i: (0, 0)),
        out_shape=jax.ShapeDtypeStruct((1, OUT), jnp.float32),
        scratch_shapes=[pltpu.VMEM((1, H), jnp.float32)],
    )(z_ch, w1.astype(jnp.bfloat16), b1.reshape(1, H),
      w2.astype(jnp.bfloat16), b2.reshape(1, H), wh, bh.reshape(1, OUT))


@jax.jit
def kernel(x, edge_index, params):
    ept = E // NUM_TILES
    srcr = edge_index[0].reshape(NUM_TILES, ept)
    dstr = edge_index[1].reshape(NUM_TILES, ept)
    pad_dst = N + jnp.broadcast_to(jnp.arange(PADN, dtype=jnp.int32) % DUMP,
                                   (NUM_TILES, PADN))
    pad_src = jnp.broadcast_to(jnp.arange(PADN, dtype=jnp.int32),
                               (NUM_TILES, PADN))
    src = jnp.concatenate([srcr, pad_src], axis=1).reshape(NUM_TILES * BPT, EB)
    dst = jnp.concatenate([dstr, pad_dst], axis=1).reshape(NUM_TILES * BPT, EB)
    L = len(params["convs"])

    IN = x.shape[1]
    h = jnp.moveaxis(x.reshape(N, IN // CHUNK, CHUNK), 1, 0)

    out = None
    for l in range(L):
        p = params["convs"][l]
        z = _sc_agg(h, src, dst)
        if l < L - 1:
            h = _tc_mlp(z, p["W1"], p["b1"], p["W2"], p["b2"], relu_out=True)
        else:
            hp = params["head"]
            out = _tc_mlp_head(z, p["W1"], p["b1"], p["W2"], p["b2"],
                               hp["Wh"], hp["bh"])
    return out


# speedup vs baseline: 1.1647x; 1.0103x over previous
import functools

import jax
import jax.numpy as jnp
from jax import lax
from jax.experimental import pallas as pl
from jax.experimental.pallas import tpu as pltpu
from jax.experimental.pallas import tpu_sc as plsc

N = 10000
E = 160000
CHUNK = 128
EB = 128
NUM_SC = 2
NUM_TILES = 16
BPT = 80
EP = NUM_TILES * BPT * EB
PADN = BPT * EB - E // NUM_TILES
DUMP = 16
NPAD = N + DUMP
HBPT = BPT // 2


def _sc_agg(h_ch, src, dst):
    C = h_ch.shape[0]
    chunks_per_core = C // NUM_SC
    mesh = plsc.VectorSubcoreMesh(core_axis_name="c", subcore_axis_name="s")

    @functools.partial(
        pl.kernel,
        out_type=jax.ShapeDtypeStruct((C, N, CHUNK), jnp.float32),
        mesh=mesh,
        scratch_types=[
            pltpu.VMEM_SHARED((NPAD, CHUNK), jnp.float32),
            pltpu.VMEM((HBPT, EB), jnp.int32),
            pltpu.VMEM((HBPT, EB), jnp.int32),
            pltpu.VMEM((EB, CHUNK), jnp.float32),
            pltpu.VMEM((EB, CHUNK), jnp.float32),
            pltpu.SemaphoreType.DMA,
            pltpu.SemaphoreType.DMA,
            pltpu.SemaphoreType.DMA,
            pltpu.SemaphoreType.DMA,
        ],
    )
    def agg_kernel(h_hbm, src_hbm, dst_hbm, z_hbm, acc,
                   srcall, dstall, r0, r1, g0, g1, s0, s1):
        rows = [r0, r1]
        gsem = [g0, g1]
        ssem = [s0, s1]
        cid = lax.axis_index("c")
        sid = lax.axis_index("s")
        rpt = 624
        tail0 = rpt * NUM_TILES
        tail = N - tail0
        row0 = sid * rpt

        for chunk in range(C):
            owner = chunk // chunks_per_core
            hc = h_hbm.at[chunk]

            @pl.when(cid == owner)
            def _():
                pltpu.sync_copy(
                    h_hbm.at[chunk, pl.ds(row0, rpt)],
                    acc.at[pl.ds(row0, rpt)],
                )

                @pl.when(sid == 0)
                def _():
                    pltpu.sync_copy(
                        h_hbm.at[chunk, pl.ds(tail0, tail)],
                        acc.at[pl.ds(tail0, tail)],
                    )

                plsc.subcore_barrier()

                def start_gather(k, b):
                    pltpu.async_copy(hc.at[srcall.at[k]], rows[b], gsem[b])

                def wait_gather(k, b):
                    pltpu.make_async_copy(hc.at[srcall.at[k]], rows[b],
                                          gsem[b]).wait()

                def start_scatter(k, b):
                    pltpu.async_copy(rows[b], acc.at[dstall.at[k]], ssem[b],
                                     add=True)

                def wait_scatter(k, b):
                    pltpu.make_async_copy(rows[b], acc.at[dstall.at[k]],
                                          ssem[b]).wait()

                for half in range(2):
                    pltpu.sync_copy(
                        src_hbm.at[pl.ds(sid * BPT + half * HBPT, HBPT)],
                        srcall)
                    pltpu.sync_copy(
                        dst_hbm.at[pl.ds(sid * BPT + half * HBPT, HBPT)],
                        dstall)
                    start_gather(0, 0)

                    @pl.loop(0, HBPT, step=2)
                    def _(k):
                        @pl.when(k > 0)
                        def _():
                            wait_scatter(k - 1, 1)

                        start_gather(k + 1, 1)
                        wait_gather(k, 0)
                        start_scatter(k, 0)

                        @pl.when(k + 2 < HBPT)
                        def _():
                            wait_scatter(k, 0)
                            start_gather(k + 2, 0)

                        wait_gather(k + 1, 1)
                        start_scatter(k + 1, 1)

                    wait_scatter(HBPT - 2, 0)
                    wait_scatter(HBPT - 1, 1)

                plsc.subcore_barrier()

                pltpu.sync_copy(
                    acc.at[pl.ds(row0, rpt)],
                    z_hbm.at[chunk, pl.ds(row0, rpt)],
                )

                @pl.when(sid == 0)
                def _():
                    pltpu.sync_copy(
                        acc.at[pl.ds(tail0, tail)],
                        z_hbm.at[chunk, pl.ds(tail0, tail)],
                    )

                plsc.subcore_barrier()

    return agg_kernel(h_ch, src, dst)


def _mlp_body(z_ref, w1_ref, b1_ref, w2_ref, b2_ref, out_ref, *, relu_out):
    zc = z_ref[...]
    z = jnp.concatenate([zc[c] for c in range(zc.shape[0])], axis=-1)
    a = jnp.maximum(
        jnp.dot(z.astype(jnp.bfloat16), w1_ref[...],
                preferred_element_type=jnp.float32) + b1_ref[...],
        0.0,
    )
    y = jnp.dot(a.astype(jnp.bfloat16), w2_ref[...],
                preferred_element_type=jnp.float32) + b2_ref[...]
    if relu_out:
        y = jnp.maximum(y, 0.0)
    for c in range(out_ref.shape[0]):
        out_ref[c] = y[:, c * CHUNK:(c + 1) * CHUNK]


def _tc_mlp(z_ch, w1, b1, w2, b2, *, bn=2000, relu_out=True):
    C = z_ch.shape[0]
    H = w2.shape[1]
    Co = H // CHUNK
    grid = (N // bn,)
    return pl.pallas_call(
        functools.partial(_mlp_body, relu_out=relu_out),
        grid=grid,
        in_specs=[
            pl.BlockSpec((C, bn, CHUNK), lambda i: (0, i, 0)),
            pl.BlockSpec((C * CHUNK, H), lambda i: (0, 0)),
            pl.BlockSpec((1, H), lambda i: (0, 0)),
            pl.BlockSpec((H, H), lambda i: (0, 0)),
            pl.BlockSpec((1, H), lambda i: (0, 0)),
        ],
        out_specs=pl.BlockSpec((Co, bn, CHUNK), lambda i: (0, i, 0)),
        out_shape=jax.ShapeDtypeStruct((Co, N, CHUNK), jnp.float32),
    )(z_ch, w1.astype(jnp.bfloat16), b1.reshape(1, H),
      w2.astype(jnp.bfloat16), b2.reshape(1, H))


def _head_body(z_ref, w1_ref, b1_ref, w2_ref, b2_ref, wh_ref, bh_ref,
               out_ref, acc_ref):
    i = pl.program_id(0)
    zc = z_ref[...]
    z = jnp.concatenate([zc[c] for c in range(zc.shape[0])], axis=-1)
    a = jnp.maximum(
        jnp.dot(z.astype(jnp.bfloat16), w1_ref[...],
                preferred_element_type=jnp.float32) + b1_ref[...],
        0.0,
    )
    y = jnp.dot(a.astype(jnp.bfloat16), w2_ref[...],
                preferred_element_type=jnp.float32) + b2_ref[...]
    colsum = jnp.sum(y, axis=0, keepdims=True)

    @pl.when(i == 0)
    def _():
        acc_ref[...] = jnp.zeros_like(acc_ref)

    acc_ref[...] += colsum
    out_ref[...] = (
        jnp.dot(acc_ref[...] / N, wh_ref[...], preferred_element_type=jnp.float32)
        + bh_ref[...]
    )


def _tc_mlp_head(z_ch, w1, b1, w2, b2, wh, bh, *, bn=2000):
    C = z_ch.shape[0]
    H = w2.shape[1]
    OUT = wh.shape[1]
    grid = (N // bn,)
    return pl.pallas_call(
        _head_body,
        grid=grid,
        in_specs=[
            pl.BlockSpec((C, bn, CHUNK), lambda i: (0, i, 0)),
            pl.BlockSpec((C * CHUNK, H), lambda i: (0, 0)),
            pl.BlockSpec((1, H), lambda i: (0, 0)),
            pl.BlockSpec((H, H), lambda i: (0, 0)),
            pl.BlockSpec((1, H), lambda i: (0, 0)),
            pl.BlockSpec((H, OUT), lambda i: (0, 0)),
            pl.BlockSpec((1, OUT), lambda i: (0, 0)),
        ],
        out_specs=pl.BlockSpec((1, OUT), lambda i: (0, 0)),
        out_shape=jax.ShapeDtypeStruct((1, OUT), jnp.float32),
        scratch_shapes=[pltpu.VMEM((1, H), jnp.float32)],
    )(z_ch, w1.astype(jnp.bfloat16), b1.reshape(1, H),
      w2.astype(jnp.bfloat16), b2.reshape(1, H), wh, bh.reshape(1, OUT))


@jax.jit
def kernel(x, edge_index, params):
    ept = E // NUM_TILES
    srcr = edge_index[0].reshape(NUM_TILES, ept)
    dstr = edge_index[1].reshape(NUM_TILES, ept)
    pad_dst = N + jnp.broadcast_to(jnp.arange(PADN, dtype=jnp.int32) % DUMP,
                                   (NUM_TILES, PADN))
    pad_src = jnp.broadcast_to(jnp.arange(PADN, dtype=jnp.int32),
                               (NUM_TILES, PADN))
    src = jnp.concatenate([srcr, pad_src], axis=1).reshape(NUM_TILES * BPT, EB)
    dst = jnp.concatenate([dstr, pad_dst], axis=1).reshape(NUM_TILES * BPT, EB)
    L = len(params["convs"])

    IN = x.shape[1]
    h = jnp.moveaxis(x.reshape(N, IN // CHUNK, CHUNK), 1, 0)

    out = None
    for l in range(L):
        p = params["convs"][l]
        z = _sc_agg(h, src, dst)
        if l < L - 1:
            h = _tc_mlp(z, p["W1"], p["b1"], p["W2"], p["b2"], relu_out=True)
        else:
            hp = params["head"]
            out = _tc_mlp_head(z, p["W1"], p["b1"], p["W2"], p["b2"],
                               hp["Wh"], hp["bh"])
    return out
